# baseline (speedup 1.0000x reference)
import functools

import jax
import jax.numpy as jnp
from jax import lax
from jax.experimental import pallas as pl
from jax.experimental.pallas import tpu as pltpu
from jax.experimental.pallas import tpu_sc as plsc

_N_USERS = 1_000_000
_EMB = 64
_B = 16384
_K_NEG = 4
_DECAY = 1e-4

_NC = 2
_NS = 16
_NW = _NC * _NS
_BPW = _B // _NW
_LANES = 16
_Q = 128
_NQ = _BPW // _Q
_CPQ = _Q // _LANES


def _sc_body(tbl, user, pos, negt, diff_out, sq_out,
             idxv, par, big, u_col, c_col, diff_buf, sq_buf, sem):
    wid = lax.axis_index("s") * _NC + lax.axis_index("c")
    base = wid * _BPW
    lane = jnp.arange(_LANES, dtype=jnp.int32)

    sq = jnp.zeros((_LANES,), jnp.float32)

    for a in range(6):
        if a == 0:
            src = user.at[pl.ds(base, _BPW)]
            off = 0
        elif a == 1:
            src = pos.at[pl.ds(base, _BPW)]
            off = _N_USERS
        else:
            src = negt.at[a - 2, pl.ds(base, _BPW)]
            off = _N_USERS

        pltpu.sync_copy(src, idxv)

        def _rows(i, c, off=off):
            s = pl.ds(i * _LANES, _LANES)
            raw = idxv[s]
            par[s] = (raw & 1) * _EMB
            idxv[s] = (raw + off) >> 1
            return c
        lax.fori_loop(0, _BPW // _LANES, _rows, 0)

        for q in range(_NQ):
            pltpu.async_copy(
                tbl.at[idxv.at[pl.ds(q * _Q, _Q)]], big, sem
            ).wait()

            def _proc(d, s2, q=q, a=a):
                for c in range(_CPQ):
                    rows = lane + (c * _LANES)
                    cols = par[pl.ds(q * _Q + c * _LANES, _LANES)] + d
                    v = plsc.load_gather(big, [rows, cols])
                    s2 = s2 + v * v
                    dst = pl.ds(q * _Q + c * _LANES, _LANES)
                    if a == 0:
                        u_col[d, dst] = v
                    elif a == 1:
                        c_col[d, dst] = v
                    else:
                        c_col[d, dst] = c_col[d, dst] - 0.25 * v
                return s2
            sq = lax.fori_loop(0, _EMB, _proc, sq)

    def _dot(i, c):
        s = pl.ds(i * _LANES, _LANES)
        def _inner(d, acc):
            return acc + u_col[d, s] * c_col[d, s]
        diff_buf[s] = lax.fori_loop(0, _EMB, _inner,
                                    jnp.zeros((_LANES,), jnp.float32))
        return c
    lax.fori_loop(0, _BPW // _LANES, _dot, 0)

    sq_buf[:] = sq
    pltpu.sync_copy(diff_buf, diff_out.at[pl.ds(base, _BPW)])
    pltpu.sync_copy(sq_buf, sq_out.at[wid])


_sc_gather = functools.partial(
    pl.kernel,
    mesh=plsc.VectorSubcoreMesh(core_axis_name="c", subcore_axis_name="s"),
    compiler_params=pltpu.CompilerParams(
        use_tc_tiling_on_sc=False, needs_layout_passes=False
    ),
    out_type=[
        jax.ShapeDtypeStruct((_B,), jnp.float32),
        jax.ShapeDtypeStruct((_NW, _LANES), jnp.float32),
    ],
    scratch_types=[
        pltpu.VMEM((_BPW,), jnp.int32),
        pltpu.VMEM((_BPW,), jnp.int32),
        pltpu.VMEM((_Q, 2 * _EMB), jnp.float32),
        pltpu.VMEM((_EMB, _BPW), jnp.float32),
        pltpu.VMEM((_EMB, _BPW), jnp.float32),
        pltpu.VMEM((_BPW,), jnp.float32),
        pltpu.VMEM((_LANES,), jnp.float32),
        pltpu.SemaphoreType.DMA,
    ],
)(_sc_body)


def _finish_body(diff_ref, sq_ref, out_ref):
    s = diff_ref[:]
    ls = jnp.minimum(s, 0.0) - jnp.log1p(jnp.exp(-jnp.abs(s)))
    mf = -jnp.mean(ls)
    reg = jnp.sum(sq_ref[:])
    emb = _DECAY * reg * 0.5 / _B
    out_ref[0] = mf + emb
    out_ref[1] = mf
    out_ref[2] = emb


def kernel(all_embed, user, pos_item, neg_item):
    user = user.astype(jnp.int32)
    pos = pos_item.astype(jnp.int32)
    negt = neg_item.astype(jnp.int32).T
    tbl = all_embed.reshape(_N_USERS, 2 * _EMB)

    diff, sq = _sc_gather(tbl, user, pos, negt)

    out = pl.pallas_call(
        _finish_body,
        out_shape=jax.ShapeDtypeStruct((3,), jnp.float32),
        in_specs=[
            pl.BlockSpec(memory_space=pltpu.VMEM),
            pl.BlockSpec(memory_space=pltpu.VMEM),
        ],
        out_specs=pl.BlockSpec(memory_space=pltpu.SMEM),
    )(diff, sq)
    return (out[0], out[1], out[2])


# speedup vs baseline: 1.3374x; 1.3374x over previous
import functools

import jax
import jax.numpy as jnp
from jax import lax
from jax.experimental import pallas as pl
from jax.experimental.pallas import tpu as pltpu
from jax.experimental.pallas import tpu_sc as plsc

_N_USERS = 1_000_000
_EMB = 64
_B = 16384
_K_NEG = 4
_DECAY = 1e-4

_NC = 2
_NS = 16
_NW = _NC * _NS
_BPW = _B // _NW
_LANES = 16
_Q = 128
_NQ = _BPW // _Q
_CPQ = _Q // _LANES


def _sc_body(tbl, idx_all, diff_out, sq_out,
             idxv, big, u_col, c_col, diff_buf, sq_buf, sem):
    wid = lax.axis_index("s") * _NC + lax.axis_index("c")
    base = wid * _BPW
    lane = jnp.arange(_LANES, dtype=jnp.int32)

    sq = jnp.zeros((_LANES,), jnp.float32)

    for a in range(6):
        pltpu.sync_copy(idx_all.at[pl.ds(a * _B + base, _BPW)], idxv)
        half = 0 if a == 0 else _EMB

        if a > 0:
            def _shift(i, c):
                s = pl.ds(i * _LANES, _LANES)
                idxv[s] = idxv[s] + _EMB
                return c
            lax.fori_loop(0, _BPW // _LANES, _shift, 0)

        for q in range(_NQ):
            pltpu.async_copy(
                tbl.at[idxv.at[pl.ds(q * _Q, _Q)]], big, sem
            ).wait()

            def _proc(d, s2, q=q, a=a, half=half):
                for c in range(_CPQ):
                    rows = lane + (c * _LANES)
                    cols = jnp.full((_LANES,), half, jnp.int32) + d
                    v = plsc.load_gather(big, [rows, cols])
                    s2 = s2 + v * v
                    dst = pl.ds(q * _Q + c * _LANES, _LANES)
                    if a == 0:
                        u_col[d, dst] = v
                    elif a == 1:
                        c_col[d, dst] = v
                    else:
                        c_col[d, dst] = c_col[d, dst] - 0.25 * v
                return s2
            sq = lax.fori_loop(0, _EMB, _proc, sq)

    def _dot(i, c):
        s = pl.ds(i * _LANES, _LANES)
        def _inner(d, acc):
            return acc + u_col[d, s] * c_col[d, s]
        diff_buf[s] = lax.fori_loop(0, _EMB, _inner,
                                    jnp.zeros((_LANES,), jnp.float32))
        return c
    lax.fori_loop(0, _BPW // _LANES, _dot, 0)

    sq_buf[:] = sq
    pltpu.sync_copy(diff_buf, diff_out.at[pl.ds(base, _BPW)])
    pltpu.sync_copy(sq_buf, sq_out.at[pl.ds(wid * _LANES, _LANES)])


_sc_gather = functools.partial(
    pl.kernel,
    mesh=plsc.VectorSubcoreMesh(core_axis_name="c", subcore_axis_name="s"),
    compiler_params=pltpu.CompilerParams(needs_layout_passes=False),
    out_type=[
        jax.ShapeDtypeStruct((_B,), jnp.float32),
        jax.ShapeDtypeStruct((_NW * _LANES,), jnp.float32),
    ],
    scratch_types=[
        pltpu.VMEM((_BPW,), jnp.int32),
        pltpu.VMEM((_Q, 2 * _EMB), jnp.float32),
        pltpu.VMEM((_EMB, _BPW), jnp.float32),
        pltpu.VMEM((_EMB, _BPW), jnp.float32),
        pltpu.VMEM((_BPW,), jnp.float32),
        pltpu.VMEM((_LANES,), jnp.float32),
        pltpu.SemaphoreType.DMA,
    ],
)(_sc_body)


_TW = 1024
_TG = -(-_N_USERS // _TW)
_TROWS = _TG * _TW
_N2 = 2 * _N_USERS
_SUBS = _TW // 128
_TAILSHIFT = 384


def _t_in_copy(hbm_in, buf, isem, g, slot):
    us = pl.multiple_of(g * _TW, 128)
    u = pltpu.make_async_copy(
        hbm_in.at[:, pl.ds(us, _TW)], buf.at[slot, 0], isem)
    ist = jnp.where(g == _TG - 1, _N2 - _TW, _N_USERS - _EMB + g * _TW)
    ist = pl.multiple_of(ist, 128)
    i = pltpu.make_async_copy(
        hbm_in.at[:, pl.ds(ist, _TW)], buf.at[slot, 1], isem)
    return u, i


def _t_out_copy(hbm_out, obuf, osem, g, slot):
    return pltpu.make_async_copy(
        obuf.at[slot], hbm_out.at[pl.ds(g * _TW, _TW)], osem)


def _transpose_body(hbm_in, hbm_out, buf, obuf, ibuf, isem, osem):
    g = pl.program_id(0)
    slot = lax.rem(g, 2)
    nxt = lax.rem(g + 1, 2)

    @pl.when(g == 0)
    def _prime():
        for cp in _t_in_copy(hbm_in, buf, isem, 0, 0):
            cp.start()

    @pl.when(g + 1 < _TG)
    def _ahead():
        for cp in _t_in_copy(hbm_in, buf, isem, g + 1, nxt):
            cp.start()

    for cp in _t_in_copy(hbm_in, buf, isem, g, slot):
        cp.wait()

    @pl.when(g >= 2)
    def _drain_prev():
        _t_out_copy(hbm_out, obuf, osem, g - 2, slot).wait()

    @pl.when(g < _TG - 1)
    def _main():
        for kk in range(_SUBS):
            tu = buf[slot, 0, :, kk * 128:(kk + 1) * 128].T
            ti = buf[slot, 1, :, kk * 128:(kk + 1) * 128].T
            obuf[slot, kk * 128:(kk + 1) * 128, :] = (
                jnp.concatenate([tu, ti], axis=1))

    @pl.when(g == _TG - 1)
    def _tail():
        for kk in range(_SUBS):
            ibuf[kk * 128:(kk + 1) * 128, :] = (
                buf[slot, 1, :, kk * 128:(kk + 1) * 128].T)
        for kk in range(_SUBS):
            tu = buf[slot, 0, :, kk * 128:(kk + 1) * 128].T
            ti = ibuf[_TAILSHIFT + kk * 128:_TAILSHIFT + (kk + 1) * 128, :]
            obuf[slot, kk * 128:(kk + 1) * 128, :] = (
                jnp.concatenate([tu, ti], axis=1))

    _t_out_copy(hbm_out, obuf, osem, g, slot).start()

    @pl.when(g == _TG - 1)
    def _drain_last():
        _t_out_copy(hbm_out, obuf, osem, g - 1, nxt).wait()
        _t_out_copy(hbm_out, obuf, osem, g, slot).wait()


def _transpose(tbl_t):
    return pl.pallas_call(
        _transpose_body,
        grid=(_TG,),
        in_specs=[pl.BlockSpec(memory_space=pl.ANY)],
        out_specs=pl.BlockSpec(memory_space=pl.ANY),
        out_shape=jax.ShapeDtypeStruct((_TROWS, 2 * _EMB), jnp.float32),
        scratch_shapes=[
            pltpu.VMEM((2, 2, _EMB, _TW), jnp.float32),
            pltpu.VMEM((2, _TW, 2 * _EMB), jnp.float32),
            pltpu.VMEM((_TW + 2 * _TAILSHIFT, _EMB), jnp.float32),
            pltpu.SemaphoreType.DMA,
            pltpu.SemaphoreType.DMA,
        ],
    )(tbl_t)


def _finish_body(diff_ref, sq_ref, out_ref):
    s = diff_ref[:]
    ls = jnp.minimum(s, 0.0) - jnp.log1p(jnp.exp(-jnp.abs(s)))
    mf = -jnp.mean(ls)
    reg = jnp.sum(sq_ref[:])
    emb = _DECAY * reg * 0.5 / _B
    out_ref[0] = mf + emb
    out_ref[1] = mf
    out_ref[2] = emb


def kernel(all_embed, user, pos_item, neg_item):
    user = user.astype(jnp.int32)
    pos = pos_item.astype(jnp.int32)
    negt = neg_item.astype(jnp.int32).T.reshape(-1)
    idx_all = jnp.concatenate([user, pos, negt])
    tbl = _transpose(all_embed.T)

    diff, sq = _sc_gather(tbl, idx_all)

    out = pl.pallas_call(
        _finish_body,
        out_shape=jax.ShapeDtypeStruct((3,), jnp.float32),
        in_specs=[
            pl.BlockSpec(memory_space=pltpu.VMEM),
            pl.BlockSpec(memory_space=pltpu.VMEM),
        ],
        out_specs=pl.BlockSpec(memory_space=pltpu.SMEM),
    )(diff, sq)
    return (out[0], out[1], out[2])
